# Initial kernel scaffold; baseline (speedup 1.0000x reference)
import jax
import jax.numpy as jnp
from jax import lax
from jax.experimental import pallas as pl
from jax.experimental.pallas import tpu as pltpu
from jax.experimental.pallas import tpu_sc as plsc

_N = 10000
_E = 320000
_D = 128
_DE = 16
_C = 128
_G = 64

_NC = 2
_NS = 16
_NW = _NC * _NS
_EPT = _E // _NW
_CH = 80
_NCHUNK = _EPT // _CH
_RPT = _N // _NS
_RZ = 125


def _pre_body(x_ref, wg_ref, asrc_ref, adst_ref, h_ref, as_ref, ad_ref):
    h = jnp.dot(x_ref[...], wg_ref[...], preferred_element_type=jnp.float32)
    h_ref[...] = h
    as_ref[...] = jnp.sum(h * asrc_ref[...], axis=1)
    ad_ref[...] = jnp.sum(h * adst_ref[...], axis=1)


def _sc_body(h_hbm, as_hbm, ad_hbm, src_hbm, dst_hbm, ea_hbm,
             acc_out, den_out, s_out, cnt_out,
             as_v, ad_v, src_v, dst_v, w_v, wrow_v, ea_v, crow_v, rows_v,
             z128_v, z16_v, acc_sh, den_sh, s_sh, cnt_sh):
    cid = lax.axis_index("c")
    sid = lax.axis_index("s")
    wid = cid * _NS + sid

    zf = jnp.zeros((16,), jnp.float32)
    e0 = jnp.where(lax.iota(jnp.int32, 16) == 0, 1.0, 0.0)

    @pl.loop(0, _RZ)
    def _(r):
        @pl.loop(0, _D // 16)
        def _(k):
            z128_v[r, pl.ds(k * 16, 16)] = zf

    @pl.loop(0, _RPT)
    def _(r):
        z16_v[r, :] = zf

    @pl.loop(0, _CH)
    def _(r):
        wrow_v[r, :] = zf
        crow_v[r, :] = e0

    base = sid * _RPT

    @pl.loop(0, _RPT // _RZ)
    def _(b):
        pltpu.sync_copy(z128_v, acc_sh.at[pl.ds(base + b * _RZ, _RZ)])

    pltpu.sync_copy(z16_v, den_sh.at[pl.ds(base, _RPT)])
    pltpu.sync_copy(z16_v, s_sh.at[pl.ds(base, _RPT)])
    pltpu.sync_copy(z16_v, cnt_sh.at[pl.ds(base, _RPT)])

    pltpu.sync_copy(as_hbm, as_v)
    pltpu.sync_copy(ad_hbm, ad_v)

    plsc.subcore_barrier()

    ebase = wid * _EPT

    @pl.loop(0, _NCHUNK)
    def _(cnk):
        off = ebase + cnk * _CH
        pltpu.sync_copy(src_hbm.at[pl.ds(off, _CH)], src_v)
        pltpu.sync_copy(dst_hbm.at[pl.ds(off, _CH)], dst_v)
        pltpu.sync_copy(ea_hbm.at[pl.ds(off, _CH)], ea_v)

        @pl.loop(0, _CH // 16)
        def _(j):
            sv = src_v[pl.ds(j * 16, 16)]
            dv = dst_v[pl.ds(j * 16, 16)]
            e = plsc.load_gather(as_v, [sv]) + plsc.load_gather(ad_v, [dv])
            e = jnp.where(e >= 0, e, 0.2 * e)
            w = jnp.exp(e)
            w_v[pl.ds(j * 16, 16)] = w
            ridx = j * 16 + lax.iota(jnp.int32, 16)
            plsc.store_scatter(wrow_v, [ridx, jnp.zeros((16,), jnp.int32)], w)

        pltpu.sync_copy(h_hbm.at[src_v], rows_v)

        @pl.loop(0, _CH)
        def _(r):
            wb = plsc.load_gather(w_v, [jnp.broadcast_to(r, (16,))])

            @pl.loop(0, _D // 16)
            def _(k):
                rows_v[r, pl.ds(k * 16, 16)] = rows_v[r, pl.ds(k * 16, 16)] * wb

        pltpu.sync_copy(rows_v, acc_sh.at[dst_v], add=True)
        pltpu.sync_copy(wrow_v, den_sh.at[dst_v], add=True)
        pltpu.sync_copy(ea_v, s_sh.at[src_v], add=True)
        pltpu.sync_copy(crow_v, cnt_sh.at[src_v], add=True)

    plsc.subcore_barrier()

    pltpu.sync_copy(acc_sh.at[pl.ds(base, _RPT)],
                    acc_out.at[cid, pl.ds(base, _RPT)])
    pltpu.sync_copy(den_sh.at[pl.ds(base, _RPT)],
                    den_out.at[cid, pl.ds(base, _RPT)])
    pltpu.sync_copy(s_sh.at[pl.ds(base, _RPT)],
                    s_out.at[cid, pl.ds(base, _RPT)])
    pltpu.sync_copy(cnt_sh.at[pl.ds(base, _RPT)],
                    cnt_out.at[cid, pl.ds(base, _RPT)])


def _post_body(acc_ref, den_ref, s_ref, cnt_ref, h_ref, as_ref, ad_ref,
               batch_ref, bgat_ref, we_ref, bedge_ref, w1_ref, b1_ref,
               w2_ref, b2_ref, w3_ref, b3_ref, out_ref):
    a = as_ref[...] + ad_ref[...]
    wself = jnp.exp(jnp.where(a >= 0, a, 0.2 * a))
    den = jnp.sum(den_ref[0] + den_ref[1], axis=1) + wself
    cnt = jnp.sum(cnt_ref[0] + cnt_ref[1], axis=1)
    h = h_ref[...]
    acc = acc_ref[0] + acc_ref[1] + wself[:, None] * h
    node_feat = acc / (den[:, None] + 1e-16) + bgat_ref[...][None, :]
    s = s_ref[0] + s_ref[1]
    edge_agg = (jnp.dot(s, we_ref[...], preferred_element_type=jnp.float32)
                + cnt[:, None] * bedge_ref[...][None, :])
    combined = node_feat + edge_agg
    b = batch_ref[...]
    p = (jnp.broadcast_to(b[None, :], (_G, _N))
         == lax.broadcasted_iota(jnp.int32, (_G, _N), 0)).astype(jnp.float32)
    pooled = jnp.dot(p, combined, preferred_element_type=jnp.float32)
    hm = jnp.maximum(
        jnp.dot(pooled, w1_ref[...], preferred_element_type=jnp.float32)
        + b1_ref[...][None, :], 0.0)
    hm = jnp.maximum(
        jnp.dot(hm, w2_ref[...], preferred_element_type=jnp.float32)
        + b2_ref[...][None, :], 0.0)
    out_ref[...] = (jnp.dot(hm, w3_ref[...], preferred_element_type=jnp.float32)
                    + b3_ref[...][None, :])


_f32 = jnp.float32

_pre_call = pl.pallas_call(
    _pre_body,
    out_shape=[
        jax.ShapeDtypeStruct((_N, _D), _f32),
        jax.ShapeDtypeStruct((_N,), _f32),
        jax.ShapeDtypeStruct((_N,), _f32),
    ],
)

_sc_mesh = plsc.VectorSubcoreMesh(core_axis_name="c", subcore_axis_name="s")

_sc_call = pl.kernel(
    _sc_body,
    out_type=[
        jax.ShapeDtypeStruct((_NC, _N, _D), _f32),
        jax.ShapeDtypeStruct((_NC, _N, 16), _f32),
        jax.ShapeDtypeStruct((_NC, _N, 16), _f32),
        jax.ShapeDtypeStruct((_NC, _N, 16), _f32),
    ],
    mesh=_sc_mesh,
    scratch_types=[
        pltpu.VMEM((_N,), _f32),
        pltpu.VMEM((_N,), _f32),
        pltpu.VMEM((_CH,), jnp.int32),
        pltpu.VMEM((_CH,), jnp.int32),
        pltpu.VMEM((_CH,), _f32),
        pltpu.VMEM((_CH, 16), _f32),
        pltpu.VMEM((_CH, _DE), _f32),
        pltpu.VMEM((_CH, 16), _f32),
        pltpu.VMEM((_CH, _D), _f32),
        pltpu.VMEM((_RZ, _D), _f32),
        pltpu.VMEM((_RPT, 16), _f32),
        pltpu.VMEM_SHARED((_N, _D), _f32),
        pltpu.VMEM_SHARED((_N, 16), _f32),
        pltpu.VMEM_SHARED((_N, 16), _f32),
        pltpu.VMEM_SHARED((_N, 16), _f32),
    ],
)

_post_call = pl.pallas_call(
    _post_body,
    out_shape=jax.ShapeDtypeStruct((_G, 2), _f32),
)


def kernel(x, edge_index, edge_attr, batch, W_gat, att_src, att_dst, b_gat,
           W_edge, b_edge, W1, b1, W2, b2, W3, b3):
    h, a_s, a_d = _pre_call(x, W_gat, att_src, att_dst)
    src = edge_index[0]
    dst = edge_index[1]
    acc, den, s, cnt = _sc_call(h, a_s, a_d, src, dst, edge_attr)
    return _post_call(acc, den, s, cnt, h, a_s, a_d, batch, b_gat,
                      W_edge, b_edge, W1, b1, W2, b2, W3, b3)


# speedup vs baseline: 8.1988x; 8.1988x over previous
import dataclasses

import jax
import jax.numpy as jnp
from jax import lax
from jax.experimental import pallas as pl
from jax.experimental.pallas import tpu as pltpu
from jax.experimental.pallas import tpu_sc as plsc

_N = 10000
_E = 320000
_D = 128
_DE = 16
_C = 128
_G = 64

_NC = 2
_NS = 16
_DH = _D // 2
_EPT = _E // _NS
_CH = 80
_NCHUNK = _EPT // _CH
_ZB = 640
_ZSTRIDE = 624
_RZ = 128
_BN = 200
_NB = _N // _BN


def _round_to_bf16(x):
    u = jax.lax.bitcast_convert_type(x, jnp.uint32)
    r = (u + jnp.uint32(0x7FFF) + ((u >> 16) & jnp.uint32(1))) & jnp.uint32(0xFFFF0000)
    return jax.lax.bitcast_convert_type(r, jnp.float32)


def _pre_body(x_ref, wg_ref, asrc_ref, adst_ref, h_ref, h0_ref, h1_ref,
              as_ref, ad_ref):
    h = jnp.dot(x_ref[...], wg_ref[...], preferred_element_type=jnp.float32)
    h_ref[...] = h
    h0_ref[...] = h[:, :_DH]
    h1_ref[...] = h[:, _DH:]
    as_ref[...] = jnp.sum(h * asrc_ref[...], axis=1)
    ad_ref[...] = jnp.sum(h * adst_ref[...], axis=1)


def _sc_body(h0_hbm, h1_hbm, as_hbm, ad_hbm, src_hbm, dst_hbm, ea_hbm,
             acc_out, dc_out, s_out,
             as_v, ad_v, src_v, dst_v, w_v, wrow_v, ea_v, crow_v, rows_v,
             z128_v, z16_v, acc_sh, dc_sh, s_sh):
    cid = lax.axis_index("c")
    sid = lax.axis_index("s")

    zf = jnp.zeros((16,), jnp.float32)
    e1 = jnp.where(lax.iota(jnp.int32, 16) == 1, 1.0, 0.0)

    @pl.loop(0, _RZ)
    def _(r):
        @pl.loop(0, _DH // 16)
        def _(k):
            z128_v[r, pl.ds(k * 16, 16)] = zf

    @pl.loop(0, _ZB)
    def _(r):
        z16_v[r, :] = zf

    @pl.loop(0, _CH)
    def _(r):
        wrow_v[r, :] = zf
        crow_v[r, :] = e1

    base = sid * _ZSTRIDE

    @pl.loop(0, _ZB // _RZ)
    def _(b):
        pltpu.sync_copy(z128_v, acc_sh.at[pl.ds(base + b * _RZ, _RZ)])

    pltpu.sync_copy(z16_v, dc_sh.at[pl.ds(base, _ZB)])
    pltpu.sync_copy(z16_v, s_sh.at[pl.ds(base, _ZB)])

    pltpu.sync_copy(as_hbm, as_v)
    pltpu.sync_copy(ad_hbm, ad_v)

    plsc.subcore_barrier()

    ebase = sid * _EPT

    @pl.loop(0, _NCHUNK)
    def _(cnk):
        off = ebase + cnk * _CH
        pltpu.sync_copy(src_hbm.at[pl.ds(off, _CH)], src_v)
        pltpu.sync_copy(dst_hbm.at[pl.ds(off, _CH)], dst_v)
        pltpu.sync_copy(ea_hbm.at[pl.ds(off, _CH)], ea_v)

        @pl.loop(0, _CH // 16)
        def _(j):
            sv = src_v[pl.ds(j * 16, 16)]
            dv = dst_v[pl.ds(j * 16, 16)]
            e = plsc.load_gather(as_v, [sv]) + plsc.load_gather(ad_v, [dv])
            e = jnp.where(e >= 0, e, 0.2 * e)
            w = jnp.exp(e)
            w_v[pl.ds(j * 16, 16)] = w
            ridx = j * 16 + lax.iota(jnp.int32, 16)
            plsc.store_scatter(wrow_v, [ridx, jnp.zeros((16,), jnp.int32)], w)

        @pl.when(cid == 0)
        def _():
            pltpu.sync_copy(h0_hbm.at[src_v], rows_v)

        @pl.when(cid == 1)
        def _():
            pltpu.sync_copy(h1_hbm.at[src_v], rows_v)

        @pl.loop(0, _CH)
        def _(r):
            wb = plsc.load_gather(w_v, [jnp.broadcast_to(r, (16,))])

            @pl.loop(0, _DH // 16)
            def _(k):
                rows_v[r, pl.ds(k * 16, 16)] = rows_v[r, pl.ds(k * 16, 16)] * wb

        pltpu.sync_copy(rows_v, acc_sh.at[dst_v], add=True)

        @pl.when(cid == 0)
        def _():
            pltpu.sync_copy(wrow_v, dc_sh.at[dst_v], add=True)
            pltpu.sync_copy(crow_v, dc_sh.at[src_v], add=True)

        @pl.when(cid == 1)
        def _():
            pltpu.sync_copy(ea_v, s_sh.at[src_v], add=True)

    plsc.subcore_barrier()

    pltpu.sync_copy(acc_sh.at[pl.ds(base, _ZB)],
                    acc_out.at[cid, pl.ds(base, _ZB)])
    pltpu.sync_copy(dc_sh.at[pl.ds(base, _ZB)],
                    dc_out.at[cid, pl.ds(base, _ZB)])
    pltpu.sync_copy(s_sh.at[pl.ds(base, _ZB)],
                    s_out.at[cid, pl.ds(base, _ZB)])


def _post_body(acc_ref, dc_ref, s_ref, h_ref, as_ref, ad_ref,
               batch_ref, bgat_ref, we_ref, bedge_ref, w1_ref, b1_ref,
               w2_ref, b2_ref, w3_ref, b3_ref, out_ref, pool_acc):
    i = pl.program_id(0)
    a = as_ref[:, 0] + ad_ref[:, 0]
    wself = jnp.exp(jnp.where(a >= 0, a, 0.2 * a))
    dc = dc_ref[0] + dc_ref[1]
    den = dc[:, 0] + wself
    cnt = dc[:, 1]
    h = h_ref[...]
    acc = (jnp.concatenate([acc_ref[0], acc_ref[1]], axis=1)
           + wself[:, None] * h)
    node_feat = acc / (den[:, None] + 1e-16) + bgat_ref[...][None, :]
    s = s_ref[0] + s_ref[1]
    edge_agg = (jnp.dot(s, we_ref[...], preferred_element_type=jnp.float32,
                        precision=jax.lax.Precision.HIGHEST)
                + cnt[:, None] * bedge_ref[...][None, :])
    combined = node_feat + edge_agg
    b = batch_ref[:, 0]
    p = (jnp.broadcast_to(b[None, :], (_G, _BN))
         == lax.broadcasted_iota(jnp.int32, (_G, _BN), 0)).astype(jnp.float32)
    part = jnp.dot(p, combined, preferred_element_type=jnp.float32,
                   precision=jax.lax.Precision.HIGHEST)

    @pl.when(i == 0)
    def _():
        pool_acc[...] = part

    @pl.when(i > 0)
    def _():
        pool_acc[...] = pool_acc[...] + part

    @pl.when(i == _NB - 1)
    def _():
        pooled = pool_acc[...]
        hm = jnp.maximum(
            jnp.dot(pooled, w1_ref[...], preferred_element_type=jnp.float32)
            + b1_ref[...][None, :], 0.0)
        hm = jnp.maximum(
            jnp.dot(hm, w2_ref[...], preferred_element_type=jnp.float32)
            + b2_ref[...][None, :], 0.0)
        out_ref[...] = (jnp.dot(hm, w3_ref[...],
                                preferred_element_type=jnp.float32)
                        + b3_ref[...][None, :])


_f32 = jnp.float32

_pre_call = pl.pallas_call(
    _pre_body,
    out_shape=[
        jax.ShapeDtypeStruct((_N, _D), _f32),
        jax.ShapeDtypeStruct((_N, _DH), _f32),
        jax.ShapeDtypeStruct((_N, _DH), _f32),
        jax.ShapeDtypeStruct((_N,), _f32),
        jax.ShapeDtypeStruct((_N,), _f32),
    ],
)

_sc_mesh = plsc.VectorSubcoreMesh(core_axis_name="c", subcore_axis_name="s")

_sc_cp = pltpu.CompilerParams()
if "needs_layout_passes" in pltpu.CompilerParams.__dataclass_fields__:
    _sc_cp = dataclasses.replace(_sc_cp, needs_layout_passes=False)
if "use_tc_tiling_on_sc" in pltpu.CompilerParams.__dataclass_fields__:
    _sc_cp = dataclasses.replace(_sc_cp, use_tc_tiling_on_sc=False)

_sc_call = pl.kernel(
    _sc_body,
    compiler_params=_sc_cp,
    out_type=[
        jax.ShapeDtypeStruct((_NC, _N, _DH), _f32),
        jax.ShapeDtypeStruct((_NC, _N, 16), _f32),
        jax.ShapeDtypeStruct((_NC, _N, 16), _f32),
    ],
    mesh=_sc_mesh,
    scratch_types=[
        pltpu.VMEM((_N,), _f32),
        pltpu.VMEM((_N,), _f32),
        pltpu.VMEM((_CH,), jnp.int32),
        pltpu.VMEM((_CH,), jnp.int32),
        pltpu.VMEM((_CH,), _f32),
        pltpu.VMEM((_CH, 16), _f32),
        pltpu.VMEM((_CH, _DE), _f32),
        pltpu.VMEM((_CH, 16), _f32),
        pltpu.VMEM((_CH, _DH), _f32),
        pltpu.VMEM((_RZ, _DH), _f32),
        pltpu.VMEM((_ZB, 16), _f32),
        pltpu.VMEM_SHARED((_N, _DH), _f32),
        pltpu.VMEM_SHARED((_N, 16), _f32),
        pltpu.VMEM_SHARED((_N, 16), _f32),
    ],
)

_post_call = pl.pallas_call(
    _post_body,
    grid=(_NB,),
    in_specs=[
        pl.BlockSpec((_NC, _BN, _DH), lambda i: (0, i, 0)),
        pl.BlockSpec((_NC, _BN, 16), lambda i: (0, i, 0)),
        pl.BlockSpec((_NC, _BN, 16), lambda i: (0, i, 0)),
        pl.BlockSpec((_BN, _D), lambda i: (i, 0)),
        pl.BlockSpec((_BN, 1), lambda i: (i, 0)),
        pl.BlockSpec((_BN, 1), lambda i: (i, 0)),
        pl.BlockSpec((_BN, 1), lambda i: (i, 0)),
        pl.BlockSpec((_D,), lambda i: (0,)),
        pl.BlockSpec((_DE, _D), lambda i: (0, 0)),
        pl.BlockSpec((_D,), lambda i: (0,)),
        pl.BlockSpec((_D, 100), lambda i: (0, 0)),
        pl.BlockSpec((100,), lambda i: (0,)),
        pl.BlockSpec((100, 25), lambda i: (0, 0)),
        pl.BlockSpec((25,), lambda i: (0,)),
        pl.BlockSpec((25, 2), lambda i: (0, 0)),
        pl.BlockSpec((2,), lambda i: (0,)),
    ],
    out_specs=pl.BlockSpec((_G, 2), lambda i: (0, 0)),
    out_shape=jax.ShapeDtypeStruct((_G, 2), _f32),
    scratch_shapes=[pltpu.VMEM((_G, _D), _f32)],
)

def kernel(x, edge_index, edge_attr, batch, W_gat, att_src, att_dst, b_gat,
           W_edge, b_edge, W1, b1, W2, b2, W3, b3):
    h, h0, h1, a_s, a_d = _pre_call(x, W_gat, att_src, att_dst)
    src = edge_index[0]
    dst = edge_index[1]
    ea_r = _round_to_bf16(edge_attr)
    we_r = _round_to_bf16(W_edge)
    acc, dc, s = _sc_call(h0, h1, a_s, a_d, src, dst, ea_r)
    return _post_call(acc, dc, s, h, a_s.reshape(_N, 1), a_d.reshape(_N, 1),
                      batch.reshape(_N, 1), b_gat, we_r, b_edge,
                      W1, b1, W2, b2, W3, b3)


# speedup vs baseline: 10.3925x; 1.2676x over previous
import dataclasses

import jax
import jax.numpy as jnp
from jax import lax
from jax.experimental import pallas as pl
from jax.experimental.pallas import tpu as pltpu
from jax.experimental.pallas import tpu_sc as plsc

_N = 10000
_E = 320000
_D = 128
_DE = 16
_C = 128
_G = 64

_NC = 2
_NS = 16
_DH = _D // 2
_EPT = _E // _NS
_CH = 80
_NCHUNK = _EPT // _CH
_ZB = 640
_ZSTRIDE = 624
_RZ = 128
_BN = 200
_NB = _N // _BN


def _round_to_bf16(x):
    u = jax.lax.bitcast_convert_type(x, jnp.uint32)
    r = (u + jnp.uint32(0x7FFF) + ((u >> 16) & jnp.uint32(1))) & jnp.uint32(0xFFFF0000)
    return jax.lax.bitcast_convert_type(r, jnp.float32)


def _pre_body(x_ref, wg_ref, asrc_ref, adst_ref, h_ref, h0_ref, h1_ref,
              as_ref, ad_ref):
    h = jnp.dot(x_ref[...], wg_ref[...], preferred_element_type=jnp.float32)
    h_ref[...] = h
    h0_ref[...] = h[:, :_DH]
    h1_ref[...] = h[:, _DH:]
    as_ref[...] = jnp.sum(h * asrc_ref[...], axis=1)
    ad_ref[...] = jnp.sum(h * adst_ref[...], axis=1)


def _sc_body(h0_hbm, h1_hbm, as_hbm, ad_hbm, src_hbm, dst_hbm, ea_hbm,
             acc_out, dc_out, s_out,
             as_v, ad_v, src_v, dst_v, w_v, wrow_v, ea_v, crow_v, rows_v,
             z128_v, z16_v, acc_sh, dc_sh, s_sh, sem_s, sem_de, sem_g, sem_sc):
    cid = lax.axis_index("c")
    sid = lax.axis_index("s")

    zf = jnp.zeros((16,), jnp.float32)
    e1 = jnp.where(lax.iota(jnp.int32, 16) == 1, 1.0, 0.0)

    @pl.loop(0, _RZ)
    def _(r):
        @pl.loop(0, _DH // 16)
        def _(k):
            z128_v[r, pl.ds(k * 16, 16)] = zf

    @pl.loop(0, _ZB)
    def _(r):
        z16_v[r, :] = zf

    @pl.loop(0, _CH)
    def _(r):
        wrow_v[r, :] = zf
        crow_v[r, :] = e1

    base = sid * _ZSTRIDE

    @pl.loop(0, _ZB // _RZ)
    def _(b):
        pltpu.sync_copy(z128_v, acc_sh.at[pl.ds(base + b * _RZ, _RZ)])

    pltpu.sync_copy(z16_v, dc_sh.at[pl.ds(base, _ZB)])
    pltpu.sync_copy(z16_v, s_sh.at[pl.ds(base, _ZB)])

    pltpu.sync_copy(as_hbm, as_v)
    pltpu.sync_copy(ad_hbm, ad_v)

    plsc.subcore_barrier()

    ebase = sid * _EPT

    @pl.loop(0, _NCHUNK)
    def _(cnk):
        off = ebase + cnk * _CH
        cp_s = pltpu.async_copy(src_hbm.at[pl.ds(off, _CH)], src_v, sem_s)
        cp_d = pltpu.async_copy(dst_hbm.at[pl.ds(off, _CH)], dst_v, sem_de)
        cp_e = pltpu.async_copy(ea_hbm.at[pl.ds(off, _CH)], ea_v, sem_de)
        cp_s.wait()

        @pl.when(cid == 0)
        def _():
            pltpu.async_copy(h0_hbm.at[src_v], rows_v, sem_g)

        @pl.when(cid == 1)
        def _():
            pltpu.async_copy(h1_hbm.at[src_v], rows_v, sem_g)

        cp_d.wait()
        cp_e.wait()

        @pl.loop(0, _CH // 16)
        def _(j):
            sv = src_v[pl.ds(j * 16, 16)]
            dv = dst_v[pl.ds(j * 16, 16)]
            e = plsc.load_gather(as_v, [sv]) + plsc.load_gather(ad_v, [dv])
            e = jnp.where(e >= 0, e, 0.2 * e)
            w = jnp.exp(e)
            w_v[pl.ds(j * 16, 16)] = w
            ridx = j * 16 + lax.iota(jnp.int32, 16)
            plsc.store_scatter(wrow_v, [ridx, jnp.zeros((16,), jnp.int32)], w)

        pltpu.make_async_copy(h0_hbm.at[pl.ds(0, _CH)], rows_v, sem_g).wait()

        @pl.loop(0, _CH)
        def _(r):
            wb = plsc.load_gather(w_v, [jnp.broadcast_to(r, (16,))])

            @pl.loop(0, _DH // 16)
            def _(k):
                rows_v[r, pl.ds(k * 16, 16)] = rows_v[r, pl.ds(k * 16, 16)] * wb

        sc_r = pltpu.async_copy(rows_v, acc_sh.at[dst_v], sem_sc, add=True)

        @pl.when(cid == 0)
        def _():
            a = pltpu.async_copy(wrow_v, dc_sh.at[dst_v], sem_sc, add=True)
            b = pltpu.async_copy(crow_v, dc_sh.at[src_v], sem_sc, add=True)
            a.wait()
            b.wait()

        @pl.when(cid == 1)
        def _():
            c = pltpu.async_copy(ea_v, s_sh.at[src_v], sem_sc, add=True)
            c.wait()

        sc_r.wait()

    plsc.subcore_barrier()

    pltpu.sync_copy(acc_sh.at[pl.ds(base, _ZB)],
                    acc_out.at[cid, pl.ds(base, _ZB)])
    pltpu.sync_copy(dc_sh.at[pl.ds(base, _ZB)],
                    dc_out.at[cid, pl.ds(base, _ZB)])
    pltpu.sync_copy(s_sh.at[pl.ds(base, _ZB)],
                    s_out.at[cid, pl.ds(base, _ZB)])


def _post_body(acc_ref, dc_ref, s_ref, h_ref, as_ref, ad_ref,
               batch_ref, bgat_ref, we_ref, bedge_ref, w1_ref, b1_ref,
               w2_ref, b2_ref, w3_ref, b3_ref, out_ref, pool_acc):
    i = pl.program_id(0)
    a = as_ref[:, 0] + ad_ref[:, 0]
    wself = jnp.exp(jnp.where(a >= 0, a, 0.2 * a))
    dc = dc_ref[0] + dc_ref[1]
    den = dc[:, 0] + wself
    cnt = dc[:, 1]
    h = h_ref[...]
    acc = (jnp.concatenate([acc_ref[0], acc_ref[1]], axis=1)
           + wself[:, None] * h)
    node_feat = acc / (den[:, None] + 1e-16) + bgat_ref[...][None, :]
    s = s_ref[0] + s_ref[1]
    edge_agg = (jnp.dot(s, we_ref[...], preferred_element_type=jnp.float32,
                        precision=jax.lax.Precision.HIGHEST)
                + cnt[:, None] * bedge_ref[...][None, :])
    combined = node_feat + edge_agg
    b = batch_ref[:, 0]
    p = (jnp.broadcast_to(b[None, :], (_G, _BN))
         == lax.broadcasted_iota(jnp.int32, (_G, _BN), 0)).astype(jnp.float32)
    part = jnp.dot(p, combined, preferred_element_type=jnp.float32,
                   precision=jax.lax.Precision.HIGHEST)

    @pl.when(i == 0)
    def _():
        pool_acc[...] = part

    @pl.when(i > 0)
    def _():
        pool_acc[...] = pool_acc[...] + part

    @pl.when(i == _NB - 1)
    def _():
        pooled = pool_acc[...]
        hm = jnp.maximum(
            jnp.dot(pooled, w1_ref[...], preferred_element_type=jnp.float32)
            + b1_ref[...][None, :], 0.0)
        hm = jnp.maximum(
            jnp.dot(hm, w2_ref[...], preferred_element_type=jnp.float32)
            + b2_ref[...][None, :], 0.0)
        out_ref[...] = (jnp.dot(hm, w3_ref[...],
                                preferred_element_type=jnp.float32)
                        + b3_ref[...][None, :])


_f32 = jnp.float32

_pre_call = pl.pallas_call(
    _pre_body,
    out_shape=[
        jax.ShapeDtypeStruct((_N, _D), _f32),
        jax.ShapeDtypeStruct((_N, _DH), _f32),
        jax.ShapeDtypeStruct((_N, _DH), _f32),
        jax.ShapeDtypeStruct((_N,), _f32),
        jax.ShapeDtypeStruct((_N,), _f32),
    ],
)

_sc_mesh = plsc.VectorSubcoreMesh(core_axis_name="c", subcore_axis_name="s")

_sc_cp = pltpu.CompilerParams()
if "needs_layout_passes" in pltpu.CompilerParams.__dataclass_fields__:
    _sc_cp = dataclasses.replace(_sc_cp, needs_layout_passes=False)
if "use_tc_tiling_on_sc" in pltpu.CompilerParams.__dataclass_fields__:
    _sc_cp = dataclasses.replace(_sc_cp, use_tc_tiling_on_sc=False)

_sc_call = pl.kernel(
    _sc_body,
    compiler_params=_sc_cp,
    out_type=[
        jax.ShapeDtypeStruct((_NC, _N, _DH), _f32),
        jax.ShapeDtypeStruct((_NC, _N, 16), _f32),
        jax.ShapeDtypeStruct((_NC, _N, 16), _f32),
    ],
    mesh=_sc_mesh,
    scratch_types=[
        pltpu.VMEM((_N,), _f32),
        pltpu.VMEM((_N,), _f32),
        pltpu.VMEM((_CH,), jnp.int32),
        pltpu.VMEM((_CH,), jnp.int32),
        pltpu.VMEM((_CH,), _f32),
        pltpu.VMEM((_CH, 16), _f32),
        pltpu.VMEM((_CH, _DE), _f32),
        pltpu.VMEM((_CH, 16), _f32),
        pltpu.VMEM((_CH, _DH), _f32),
        pltpu.VMEM((_RZ, _DH), _f32),
        pltpu.VMEM((_ZB, 16), _f32),
        pltpu.VMEM_SHARED((_N, _DH), _f32),
        pltpu.VMEM_SHARED((_N, 16), _f32),
        pltpu.VMEM_SHARED((_N, 16), _f32),
        pltpu.SemaphoreType.DMA,
        pltpu.SemaphoreType.DMA,
        pltpu.SemaphoreType.DMA,
        pltpu.SemaphoreType.DMA,
    ],
)

_post_call = pl.pallas_call(
    _post_body,
    grid=(_NB,),
    in_specs=[
        pl.BlockSpec((_NC, _BN, _DH), lambda i: (0, i, 0)),
        pl.BlockSpec((_NC, _BN, 16), lambda i: (0, i, 0)),
        pl.BlockSpec((_NC, _BN, 16), lambda i: (0, i, 0)),
        pl.BlockSpec((_BN, _D), lambda i: (i, 0)),
        pl.BlockSpec((_BN, 1), lambda i: (i, 0)),
        pl.BlockSpec((_BN, 1), lambda i: (i, 0)),
        pl.BlockSpec((_BN, 1), lambda i: (i, 0)),
        pl.BlockSpec((_D,), lambda i: (0,)),
        pl.BlockSpec((_DE, _D), lambda i: (0, 0)),
        pl.BlockSpec((_D,), lambda i: (0,)),
        pl.BlockSpec((_D, 100), lambda i: (0, 0)),
        pl.BlockSpec((100,), lambda i: (0,)),
        pl.BlockSpec((100, 25), lambda i: (0, 0)),
        pl.BlockSpec((25,), lambda i: (0,)),
        pl.BlockSpec((25, 2), lambda i: (0, 0)),
        pl.BlockSpec((2,), lambda i: (0,)),
    ],
    out_specs=pl.BlockSpec((_G, 2), lambda i: (0, 0)),
    out_shape=jax.ShapeDtypeStruct((_G, 2), _f32),
    scratch_shapes=[pltpu.VMEM((_G, _D), _f32)],
)

def kernel(x, edge_index, edge_attr, batch, W_gat, att_src, att_dst, b_gat,
           W_edge, b_edge, W1, b1, W2, b2, W3, b3):
    h, h0, h1, a_s, a_d = _pre_call(x, W_gat, att_src, att_dst)
    src = edge_index[0]
    dst = edge_index[1]
    ea_r = _round_to_bf16(edge_attr)
    we_r = _round_to_bf16(W_edge)
    acc, dc, s = _sc_call(h0, h1, a_s, a_d, src, dst, ea_r)
    return _post_call(acc, dc, s, h, a_s.reshape(_N, 1), a_d.reshape(_N, 1),
                      batch.reshape(_N, 1), b_gat, we_r, b_edge,
                      W1, b1, W2, b2, W3, b3)
